# Initial kernel scaffold; baseline (speedup 1.0000x reference)
import jax
import jax.numpy as jnp
from jax.experimental import pallas as pl


def kernel(x, edge_index, relations, relation_index, W1l, b1l, W1r, b1r, W1e, att1, bias1, W2l, b2l, W2r, b2r, W2e, att2, bias2):
    raise NotImplementedError("write your pallas kernel here")


# speedup vs baseline: 6.7702x; 6.7702x over previous
import functools

import jax
import jax.numpy as jnp
from jax import lax
from jax.experimental import pallas as pl
from jax.experimental.pallas import tpu as pltpu
from jax.experimental.pallas import tpu_sc as plsc

N = 10000
E = 160000
R = 16
CH = 8
NT = 16
EPT = E // NT
K = 80
NCHUNK = EPT // K
DUMP_TILES = 10
DROWS = N // DUMP_TILES
ZR = 200
AW = 128

_mesh = plsc.VectorSubcoreMesh(core_axis_name="c", subcore_axis_name="s")


@functools.partial(
    pl.kernel,
    mesh=_mesh,
    out_type=jax.ShapeDtypeStruct((2 * E, 16), jnp.float32),
    scratch_types=[
        pltpu.VMEM((K,), jnp.int32),
        pltpu.VMEM((K,), jnp.int32),
        pltpu.VMEM((K,), jnp.int32),
        pltpu.VMEM((K, 128), jnp.float32),
        pltpu.VMEM((K, 128), jnp.float32),
        pltpu.VMEM((K, 128), jnp.float32),
        pltpu.VMEM((K, 16), jnp.float32),
        pltpu.VMEM((128,), jnp.float32),
        pltpu.SemaphoreType.DMA,
        pltpu.SemaphoreType.DMA,
        pltpu.SemaphoreType.DMA,
    ],
)
def _sc_pass1(xl_h, xr_h, rele_h, att_h, src_h, dst_h, rel_h,
              alpha_o,
              srcv, dstv, relv, xlr, xrr, errw, accr, attv,
              sem1, sem2, sem3):
    c = lax.axis_index("c")
    s = lax.axis_index("s")
    pltpu.sync_copy(att_h.at[pl.ds(pl.multiple_of(c * 128, 8), 128)], attv)
    tile_base = s * EPT

    def chunk_body(ci, _):
        gbase = pl.multiple_of(tile_base + ci * K, 8)
        hbase = pl.multiple_of(c * E + tile_base + ci * K, 8)
        pltpu.sync_copy(src_h.at[pl.ds(hbase, K)], srcv)
        pltpu.sync_copy(dst_h.at[pl.ds(hbase, K)], dstv)
        pltpu.sync_copy(rel_h.at[pl.ds(hbase, K)], relv)
        cp1 = pltpu.async_copy(xl_h.at[srcv], xlr, sem1)
        cp2 = pltpu.async_copy(xr_h.at[dstv], xrr, sem2)
        cp3 = pltpu.async_copy(rele_h.at[relv], errw, sem3)
        cp1.wait()
        cp2.wait()
        cp3.wait()

        def group_body(g, _):
            gb = pl.multiple_of(g * 16, 16)
            for i in range(16):
                e = gb + i
                acc = jnp.zeros((16,), jnp.float32)
                for ch in range(CH):
                    sl = pl.ds(ch * 16, 16)
                    m = xlr[e, sl] + xrr[e, sl] + errw[e, sl]
                    m = jnp.maximum(m, 0.0) + 0.2 * jnp.minimum(m, 0.0)
                    acc = acc + m * attv[sl]
                accr[e] = acc
            return 0

        lax.fori_loop(0, K // 16, group_body, 0)
        obase = pl.multiple_of(c * E + gbase, 8)
        pltpu.sync_copy(accr, alpha_o.at[pl.ds(obase, K)])
        return 0

    lax.fori_loop(0, NCHUNK, chunk_body, 0)


@functools.partial(
    pl.kernel,
    mesh=_mesh,
    out_type=jax.ShapeDtypeStruct((2 * N, AW), jnp.float32),
    scratch_types=[
        pltpu.VMEM((K,), jnp.int32),
        pltpu.VMEM((K,), jnp.int32),
        pltpu.VMEM((K,), jnp.float32),
        pltpu.VMEM((K, 128), jnp.float32),
        pltpu.VMEM((K, AW), jnp.float32),
        pltpu.VMEM((ZR, AW), jnp.float32),
        pltpu.VMEM_SHARED((N, AW), jnp.float32),
        pltpu.SemaphoreType.DMA,
    ],
)
def _sc_pass2(xl_h, w_h, src_h, dst_h, zero_h,
              acc_o,
              srcv, dstv, wv, xlr, msgr, zbuf, acc_sh, sem1):
    c = lax.axis_index("c")
    s = lax.axis_index("s")
    pltpu.sync_copy(zero_h, zbuf)

    @pl.when(s < DUMP_TILES)
    def _():
        for j in range(DROWS // ZR):
            off = pl.multiple_of(s * DROWS + j * ZR, 8)
            pltpu.sync_copy(zbuf, acc_sh.at[pl.ds(off, ZR)])

    plsc.subcore_barrier()

    tile_base = s * EPT

    def chunk_body(ci, _):
        gbase = pl.multiple_of(tile_base + ci * K, 8)
        hbase = pl.multiple_of(c * E + tile_base + ci * K, 8)
        pltpu.sync_copy(src_h.at[pl.ds(hbase, K)], srcv)
        pltpu.sync_copy(dst_h.at[pl.ds(gbase, K)], dstv)
        pltpu.sync_copy(w_h.at[pl.ds(hbase, K)], wv)
        pltpu.async_copy(xl_h.at[srcv], xlr, sem1).wait()

        def group_body(g, _):
            gb = pl.multiple_of(g * 16, 16)
            w16 = wv[pl.ds(gb, 16)]
            for i in range(16):
                e = gb + i
                wb = jnp.take(w16, jnp.full((16,), i, jnp.int32))
                for ch in range(CH):
                    sl = pl.ds(ch * 16, 16)
                    msgr[e, sl] = xlr[e, sl] * wb
            return 0

        lax.fori_loop(0, K // 16, group_body, 0)
        pltpu.sync_copy(msgr, acc_sh.at[dstv], add=True)
        return 0

    lax.fori_loop(0, NCHUNK, chunk_body, 0)
    plsc.subcore_barrier()

    @pl.when(s < DUMP_TILES)
    def _():
        for j in range(DROWS // ZR):
            off = pl.multiple_of(s * DROWS + j * ZR, 8)
            dst_off = pl.multiple_of(c * N + s * DROWS + j * ZR, 8)
            pltpu.sync_copy(acc_sh.at[pl.ds(off, ZR)],
                            acc_o.at[pl.ds(dst_off, ZR)])


def _proj_kernel(x_ref, w_ref, b_ref, o_ref):
    o_ref[...] = (jnp.dot(x_ref[...], w_ref[...],
                          preferred_element_type=jnp.float32) + b_ref[...])


def _proj(x, w, b):
    n, d = x.shape
    hc = w.shape[1]
    bs = 2000
    return pl.pallas_call(
        _proj_kernel,
        grid=(n // bs,),
        in_specs=[pl.BlockSpec((bs, d), lambda i: (i, 0)),
                  pl.BlockSpec((d, hc), lambda i: (0, 0)),
                  pl.BlockSpec((1, hc), lambda i: (0, 0))],
        out_specs=pl.BlockSpec((bs, hc), lambda i: (i, 0)),
        out_shape=jax.ShapeDtypeStruct((n, hc), jnp.float32),
    )(x, w, b[None])


def _epi_kernel(a0_ref, a1_ref, x0_ref, x1_ref, n0_ref, n1_ref, d0_ref,
                d1_ref, b_ref, o_ref):
    h0 = (a0_ref[...] + n0_ref[...] * x0_ref[...]) / d0_ref[...]
    h1 = (a1_ref[...] + n1_ref[...] * x1_ref[...]) / d1_ref[...]
    o_ref[...] = 0.5 * (h0 + h1) + b_ref[...]


def _epilogue(a0, a1, x0, x1, n0, n1, d0, d1, bias):
    n, cdim = a0.shape
    bs = 2000
    bcast = lambda v: jnp.broadcast_to(v[:, None], (n, cdim))
    row = pl.BlockSpec((bs, cdim), lambda i: (i, 0))
    return pl.pallas_call(
        _epi_kernel,
        grid=(n // bs,),
        in_specs=[row] * 8 + [pl.BlockSpec((1, cdim), lambda i: (0, 0))],
        out_specs=row,
        out_shape=jax.ShapeDtypeStruct((n, cdim), jnp.float32),
    )(a0, a1, x0, x1, bcast(n0), bcast(n1), bcast(d0), bcast(d1), bias[None])


def _layer(x, src2, dst2, rel2, dst, ones2e, deg, relations, wl, bl, wr, br,
           we, att, bias, zero144):
    x_l = _proj(x, wl, bl)
    x_r = _proj(x, wr, br)
    rel_e = relations @ we
    xls = jnp.concatenate([x_l[:, :128], x_l[:, 128:]], axis=0)
    xrs = jnp.concatenate([x_r[:, :128], x_r[:, 128:]], axis=0)
    reles = jnp.concatenate([rel_e[:, :128], rel_e[:, 128:]], axis=0)
    atts = jnp.concatenate([att[0], att[1]], axis=0)

    alpha_rows = _sc_pass1(xls, xrs, reles, atts, src2, dst2, rel2)
    alpha = alpha_rows.sum(-1).reshape(2, E)

    lsum = _sc_pass2(reles, ones2e, rel2, dst, zero144).reshape(2, N, 128)
    degc = jnp.maximum(deg, 1.0)[:, None]
    mask = deg[:, None] > 0
    le0 = jnp.where(mask, lsum[0] / degc, 0.0)
    le1 = jnp.where(mask, lsum[1] / degc, 0.0)
    loop_e = jnp.concatenate([le0, le1], axis=1)

    xl3 = x_l.reshape(N, 2, 128)
    m_loop = xl3 + x_r.reshape(N, 2, 128) + loop_e.reshape(N, 2, 128)
    m_loop = jnp.maximum(m_loop, 0.0) + 0.2 * jnp.minimum(m_loop, 0.0)
    alpha_loop = (m_loop * att[None]).sum(-1)

    g = jnp.maximum(alpha.max(), alpha_loop.max())
    w = jnp.exp(alpha - g)
    w_loop = jnp.exp(alpha_loop - g)

    acc = _sc_pass2(xls, w.reshape(2 * E), src2, dst, zero144)
    acc = acc.reshape(2, N, AW)
    den0 = jax.ops.segment_sum(w[0], dst, num_segments=N)
    den1 = jax.ops.segment_sum(w[1], dst, num_segments=N)
    out = _epilogue(acc[0], acc[1],
                    xl3[:, 0], xl3[:, 1],
                    w_loop[:, 0], w_loop[:, 1],
                    den0 + w_loop[:, 0],
                    den1 + w_loop[:, 1],
                    bias)
    return out


def kernel(x, edge_index, relations, relation_index,
           W1l, b1l, W1r, b1r, W1e, att1, bias1,
           W2l, b2l, W2r, b2r, W2e, att2, bias2):
    src = edge_index[0].astype(jnp.int32)
    dst = edge_index[1].astype(jnp.int32)
    rel = relation_index.astype(jnp.int32)
    src2 = jnp.concatenate([src, src + N])
    dst2 = jnp.concatenate([dst, dst + N])
    rel2 = jnp.concatenate([rel, rel + R])
    zero144 = jnp.zeros((ZR, AW), jnp.float32)
    ones2e = jnp.ones((2 * E,), jnp.float32)
    deg = jax.ops.segment_sum(jnp.ones((E,), jnp.float32), dst,
                              num_segments=N)
    h1 = _layer(x, src2, dst2, rel2, dst, ones2e, deg, relations,
                W1l, b1l, W1r, b1r, W1e, att1, bias1, zero144)
    out = _layer(h1, src2, dst2, rel2, dst, ones2e, deg, relations,
                 W2l, b2l, W2r, b2r, W2e, att2, bias2, zero144)
    return (out, relations)
